# baseline (speedup 1.0000x reference)
import jax
import jax.numpy as jnp
from jax import lax
from jax.experimental import pallas as pl

_HID = 1024
_NE = 64
_NT = 32768
_BT = 2048


def _top2(lt, ids):
    m = jnp.max(lt, axis=0, keepdims=True)
    e = jnp.exp(lt - m)
    z = jnp.sum(e, axis=0, keepdims=True)
    s = e / z
    s1 = 1.0 / z
    i1 = jnp.min(jnp.where(s == s1, ids, _NE), axis=0, keepdims=True)
    masked = jnp.where(ids == i1, -1.0, s)
    s2 = jnp.max(masked, axis=0, keepdims=True)
    i2 = jnp.min(jnp.where(masked == s2, ids, _NE), axis=0, keepdims=True)
    denom = s1 + s2
    return (jnp.concatenate([s1 / denom, s2 / denom], axis=0),
            jnp.concatenate([i1, i2], axis=0))


def _gate_body(xa_ref, xb_ref, w_ref, val_ref, idx_ref):
    w = w_ref[...]
    dn = (((1,), (1,)), ((), ()))
    lta = lax.dot_general(w, xa_ref[...], dn, preferred_element_type=jnp.float32)
    ltb = lax.dot_general(w, xb_ref[...], dn, preferred_element_type=jnp.float32)
    ids = lax.broadcasted_iota(jnp.int32, lta.shape, 0)
    va, ia = _top2(lta, ids)
    vb, ib = _top2(ltb, ids)
    val_ref[...] = jnp.concatenate([va, vb], axis=1)
    idx_ref[...] = jnp.concatenate([ia, ib], axis=1)


def kernel(x, weight):
    vals_t, idx_t = pl.pallas_call(
        _gate_body,
        grid=(_NT // (2 * _BT),),
        in_specs=[
            pl.BlockSpec((_BT, _HID), lambda i: (2 * i, 0)),
            pl.BlockSpec((_BT, _HID), lambda i: (2 * i + 1, 0)),
            pl.BlockSpec((_NE, _HID), lambda i: (0, 0)),
        ],
        out_specs=[
            pl.BlockSpec((2, 2 * _BT), lambda i: (0, i)),
            pl.BlockSpec((2, 2 * _BT), lambda i: (0, i)),
        ],
        out_shape=[
            jax.ShapeDtypeStruct((2, _NT), jnp.float32),
            jax.ShapeDtypeStruct((2, _NT), jnp.int32),
        ],
    )(x, x, weight)
    return vals_t.T, idx_t.T
